# baseline (speedup 1.0000x reference)
import functools
import math

import jax
import jax.numpy as jnp
from jax import lax
from jax.experimental import pallas as pl
from jax.experimental.pallas import tpu as pltpu
from jax.experimental.pallas import tpu_sc as plsc

_N, _K, _H = 10000, 32, 128
_SCALE = 30.0

_BN = 200
_NBLK = _N // _BN

_CH = 128
_NW = 32
_NCH_W = (-(-(_N * _K) // (_CH * _NW)) + 7) // 8 * 8
_NCHUNK = _NCH_W * _NW
_NKP = _NCHUNK * _CH


def _gelu(x):
    return 0.5 * x * (1.0 + lax.erf(x * (1.0 / math.sqrt(2.0))))


def _ln(x, g, b):
    mu = jnp.mean(x, axis=-1, keepdims=True)
    xc = x - mu
    var = jnp.mean(xc * xc, axis=-1, keepdims=True)
    return xc * lax.rsqrt(var + 1e-5) * g + b


def _dot(a, b):
    return jnp.dot(a.astype(jnp.bfloat16), b.astype(jnp.bfloat16),
                   preferred_element_type=jnp.float32)


def _prep_body(hV_ref, W1a_ref, b1_ref, W1c_ref, A1_ref, P1_ref):
    hV = hV_ref[...]
    A1_ref[...] = _dot(hV, W1a_ref[...]) + b1_ref[...]
    P1_ref[...] = _dot(hV, W1c_ref[...])


def _prep(hV, W1a, b1, W1c):
    return pl.pallas_call(
        _prep_body,
        out_shape=[jax.ShapeDtypeStruct((_N, _H), jnp.float32),
                   jax.ShapeDtypeStruct((_N, _H), jnp.float32)],
    )(hV, W1a, b1, W1c)


def _sc_gather(tbl, idx2d):
    mesh = plsc.VectorSubcoreMesh(core_axis_name="c", subcore_axis_name="s")

    @functools.partial(
        pl.kernel, mesh=mesh,
        out_type=jax.ShapeDtypeStruct((_NKP, _H), jnp.float32),
        scratch_types=[
            pltpu.VMEM((_NCH_W, _CH), jnp.int32),
            pltpu.VMEM((_CH, _H), jnp.float32),
            pltpu.VMEM((_CH, _H), jnp.float32),
            pltpu.VMEM((_CH, _H), jnp.float32),
            pltpu.VMEM((_CH, _H), jnp.float32),
            pltpu.SemaphoreType.DMA,
            pltpu.SemaphoreType.DMA,
            pltpu.SemaphoreType.DMA,
            pltpu.SemaphoreType.DMA,
        ],
    )
    def gath(tbl_hbm, idx_hbm, out_hbm, idxs, buf0, buf1, buf2, buf3,
             sem0, sem1, sem2, sem3):
        wid = lax.axis_index("s") * 2 + lax.axis_index("c")
        c0 = wid * _NCH_W
        rowb = (buf0, buf1, buf2, buf3)
        semb = (sem0, sem1, sem2, sem3)

        pltpu.sync_copy(idx_hbm.at[pl.ds(c0, _NCH_W)], idxs)

        def fire(j, b):
            pltpu.async_copy(tbl_hbm.at[idxs.at[j]], rowb[b], semb[b])

        def store(j, b):
            pltpu.make_async_copy(tbl_hbm.at[idxs.at[j]], rowb[b],
                                  semb[b]).wait()
            off = pl.multiple_of((c0 + j) * _CH, _CH)
            pltpu.sync_copy(rowb[b], out_hbm.at[pl.ds(off, _CH)])

        for b in range(4):
            fire(b, b)

        def body(q, carry):
            base = 4 * q
            for b in range(4):
                store(base + b, b)
                fire(base + b + 4, b)
            return carry

        lax.fori_loop(0, (_NCH_W - 4) // 4, body, 0)
        for b in range(4):
            store(_NCH_W - 4 + b, b)

    return gath(tbl, idx2d)


def _pass1_body(hV_ref, hE_ref, G_ref, A1_ref, ma_ref, cw_ref, mv_ref,
                W1b_ref, W2_ref, b2_ref, W3_ref, b3_ref, g1_ref, be1_ref,
                Win_ref, bin_ref, Wout_ref, bout_ref, g2_ref, be2_ref,
                W11a_ref, b11_ref, W11c_ref,
                hV_out_ref, A2_ref, P2_ref):
    hE = hE_ref[...].reshape(_BN * _K, _H)
    t = _dot(hE, W1b_ref[...]) + G_ref[...]
    t = t.reshape(_BN, _K, _H) + A1_ref[...][:, None, :]
    m = _gelu(t).reshape(_BN * _K, _H)
    m = _gelu(_dot(m, W2_ref[...]) + b2_ref[...])
    m = _dot(m, W3_ref[...]) + b3_ref[...]
    w = (ma_ref[...] * cw_ref[...]) * (1.0 / _SCALE)
    dh = jnp.sum(m.reshape(_BN, _K, _H) * w[:, :, None], axis=1)
    x = _ln(hV_ref[...] + dh, g1_ref[...], be1_ref[...])
    f = _gelu(_dot(x, Win_ref[...]) + bin_ref[...])
    f = _dot(f, Wout_ref[...]) + bout_ref[...]
    x = _ln(x + f, g2_ref[...], be2_ref[...]) * mv_ref[...]
    hV_out_ref[...] = x
    A2_ref[...] = _dot(x, W11a_ref[...]) + b11_ref[...]
    P2_ref[...] = _dot(x, W11c_ref[...])


def _pass1(hV, hE, G1, A1, ma, cw, mv, weights):
    node = pl.BlockSpec((_BN, _H), lambda i: (i, 0))
    full = lambda s: pl.BlockSpec(s, lambda i: tuple(0 for _ in s))
    return pl.pallas_call(
        _pass1_body,
        grid=(_NBLK,),
        in_specs=[
            node,
            pl.BlockSpec((_BN, _K, _H), lambda i: (i, 0, 0)),
            pl.BlockSpec((_BN * _K, _H), lambda i: (i, 0)),
            node,
            pl.BlockSpec((_BN, _K), lambda i: (i, 0)),
            pl.BlockSpec((_BN, _K), lambda i: (i, 0)),
            pl.BlockSpec((_BN, 1), lambda i: (i, 0)),
        ] + [full(w.shape) for w in weights],
        out_specs=[node, node, node],
        out_shape=[jax.ShapeDtypeStruct((_N, _H), jnp.float32)] * 3,
        compiler_params=pltpu.CompilerParams(
            dimension_semantics=("arbitrary",)),
    )(hV, hE, G1, A1, ma, cw, mv, *weights)


def _pass2_body(hE_ref, G_ref, A2_ref,
                W11b_ref, W12_ref, b12_ref, W13_ref, b13_ref, g3_ref, be3_ref,
                out_ref):
    hE = hE_ref[...].reshape(_BN * _K, _H)
    t = _dot(hE, W11b_ref[...]) + G_ref[...]
    t = t.reshape(_BN, _K, _H) + A2_ref[...][:, None, :]
    m = _gelu(t).reshape(_BN * _K, _H)
    m = _gelu(_dot(m, W12_ref[...]) + b12_ref[...])
    m = _dot(m, W13_ref[...]) + b13_ref[...]
    out = _ln(hE + m, g3_ref[...], be3_ref[...])
    out_ref[...] = out.reshape(_BN, _K, _H)


def _pass2(hE, G2, A2, weights):
    full = lambda s: pl.BlockSpec(s, lambda i: tuple(0 for _ in s))
    return pl.pallas_call(
        _pass2_body,
        grid=(_NBLK,),
        in_specs=[
            pl.BlockSpec((_BN, _K, _H), lambda i: (i, 0, 0)),
            pl.BlockSpec((_BN * _K, _H), lambda i: (i, 0)),
            pl.BlockSpec((_BN, _H), lambda i: (i, 0)),
        ] + [full(w.shape) for w in weights],
        out_specs=pl.BlockSpec((_BN, _K, _H), lambda i: (i, 0, 0)),
        out_shape=jax.ShapeDtypeStruct((_N, _K, _H), jnp.float32),
        compiler_params=pltpu.CompilerParams(
            dimension_semantics=("arbitrary",)),
    )(hE, G2, A2, *weights)


def kernel(h_V, h_E, E_idx, mask_V, mask_attend, conf_weights, params):
    p = params
    hV = h_V.reshape(_N, _H)
    hE = h_E.reshape(_N, _K, _H)
    ma = mask_attend.reshape(_N, _K)
    cw = conf_weights.reshape(_N, _K)
    mv = mask_V.reshape(_N, 1)
    idx = E_idx.reshape(_N * _K).astype(jnp.int32)
    idx2d = jnp.pad(idx, (0, _NKP - _N * _K)).reshape(_NCHUNK, _CH)

    def rows(w):
        return w[:_H], w[_H:2 * _H], w[2 * _H:]

    W1a, W1b, W1c = rows(p['W1'][0])
    W11a, W11b, W11c = rows(p['W11'][0])
    b1 = p['W1'][1].reshape(1, _H)
    b11 = p['W11'][1].reshape(1, _H)
    W2, b2 = p['W2'][0], p['W2'][1].reshape(1, _H)
    W3, b3 = p['W3'][0], p['W3'][1].reshape(1, _H)
    W12, b12 = p['W12'][0], p['W12'][1].reshape(1, _H)
    W13, b13 = p['W13'][0], p['W13'][1].reshape(1, _H)
    Win, bin_ = p['Win'][0], p['Win'][1].reshape(1, 4 * _H)
    Wout, bout = p['Wout'][0], p['Wout'][1].reshape(1, _H)
    g1, be1 = (v.reshape(1, _H) for v in p['ln1'])
    g2, be2 = (v.reshape(1, _H) for v in p['ln2'])
    g3, be3 = (v.reshape(1, _H) for v in p['ln3'])

    A1, P1 = _prep(hV, W1a, b1, W1c)
    G1 = _sc_gather(P1, idx2d)
    hV2, A2, P2 = _pass1(
        hV, hE, G1, A1, ma, cw, mv,
        [W1b, W2, b2, W3, b3, g1, be1, Win, bin_, Wout, bout, g2, be2,
         W11a, b11, W11c])
    G2 = _sc_gather(P2, idx2d)
    hE2 = _pass2(hE, G2, A2, [W11b, W12, b12, W13, b13, g3, be3])
    return (hV2.reshape(1, _N, _H), hE2.reshape(1, _N, _K, _H))


# speedup vs baseline: 2.4222x; 2.4222x over previous
import functools
import math

import jax
import jax.numpy as jnp
from jax import lax
from jax.experimental import pallas as pl
from jax.experimental.pallas import tpu as pltpu
from jax.experimental.pallas import tpu_sc as plsc

_N, _K, _H = 10000, 32, 128
_SCALE = 30.0

_BN = 200
_NBLK = _N // _BN

_CH = 128
_NW = 32
_NCH_W = (-(-(_N * _K) // (_CH * _NW)) + 7) // 8 * 8
_NCHUNK = _NCH_W * _NW
_NKP = _NCHUNK * _CH


def _gelu(x):
    return 0.5 * x * (1.0 + lax.erf(x * (1.0 / math.sqrt(2.0))))


def _ln(x, g, b):
    mu = jnp.mean(x, axis=-1, keepdims=True)
    xc = x - mu
    var = jnp.mean(xc * xc, axis=-1, keepdims=True)
    return xc * lax.rsqrt(var + 1e-5) * g + b


def _dot(a, b):
    return jnp.dot(a.astype(jnp.bfloat16), b.astype(jnp.bfloat16),
                   preferred_element_type=jnp.float32)


def _prep_body(hV_ref, W1a_ref, b1_ref, W1c_ref, A1_ref, P1_ref):
    hV = hV_ref[...]
    A1_ref[...] = _dot(hV, W1a_ref[...]) + b1_ref[...]
    P1_ref[...] = _dot(hV, W1c_ref[...])


def _prep(hV, W1a, b1, W1c):
    return pl.pallas_call(
        _prep_body,
        out_shape=[jax.ShapeDtypeStruct((_N, _H), jnp.float32),
                   jax.ShapeDtypeStruct((_N, _H), jnp.float32)],
    )(hV, W1a, b1, W1c)


def _sc_gather(tbl, idx2d):
    mesh = plsc.VectorSubcoreMesh(core_axis_name="c", subcore_axis_name="s")

    @functools.partial(
        pl.kernel, mesh=mesh,
        out_type=jax.ShapeDtypeStruct((_NKP, _H), jnp.float32),
        scratch_types=[
            pltpu.VMEM((_NCH_W, _CH), jnp.int32),
            pltpu.VMEM((_CH, _H), jnp.float32),
            pltpu.VMEM((_CH, _H), jnp.float32),
            pltpu.VMEM_SHARED((_N, _H), jnp.float32),
            pltpu.SemaphoreType.DMA,
            pltpu.SemaphoreType.DMA,
        ],
    )
    def gath(tbl_hbm, idx_hbm, out_hbm, idxs, buf0, buf1,
             tblS, sem0, sem1):
        s = lax.axis_index("s")
        wid = s * 2 + lax.axis_index("c")
        c0 = wid * _NCH_W
        rowb = (buf0, buf1)
        semb = (sem0, sem1)

        @pl.when(s < 10)
        def _():
            r0 = pl.multiple_of(s * (_N // 10), 8)
            pltpu.sync_copy(tbl_hbm.at[pl.ds(r0, _N // 10)],
                            tblS.at[pl.ds(r0, _N // 10)])
        plsc.subcore_barrier()

        pltpu.sync_copy(idx_hbm.at[pl.ds(c0, _NCH_W)], idxs)

        def fire(j, b):
            pltpu.async_copy(tblS.at[idxs.at[j]], rowb[b], semb[b])

        def store(j, b):
            pltpu.make_async_copy(tblS.at[idxs.at[j]], rowb[b],
                                  semb[b]).wait()
            off = pl.multiple_of((c0 + j) * _CH, _CH)
            pltpu.sync_copy(rowb[b], out_hbm.at[pl.ds(off, _CH)])

        for b in range(2):
            fire(b, b)

        def body(q, carry):
            base = 2 * q
            for b in range(2):
                store(base + b, b)
                fire(base + b + 2, b)
            return carry

        lax.fori_loop(0, (_NCH_W - 2) // 2, body, 0)
        for b in range(2):
            store(_NCH_W - 2 + b, b)

    return gath(tbl, idx2d)


def _pass1_body(hV_ref, hE_ref, G_ref, A1_ref, ma_ref, cw_ref, mv_ref,
                W1b_ref, W2_ref, b2_ref, W3_ref, b3_ref, g1_ref, be1_ref,
                Win_ref, bin_ref, Wout_ref, bout_ref, g2_ref, be2_ref,
                W11a_ref, b11_ref, W11c_ref,
                hV_out_ref, A2_ref, P2_ref):
    hE = hE_ref[...].reshape(_BN * _K, _H)
    t = _dot(hE, W1b_ref[...]) + G_ref[...]
    t = t.reshape(_BN, _K, _H) + A1_ref[...][:, None, :]
    m = _gelu(t).reshape(_BN * _K, _H)
    m = _gelu(_dot(m, W2_ref[...]) + b2_ref[...])
    m = _dot(m, W3_ref[...]) + b3_ref[...]
    w = (ma_ref[...] * cw_ref[...]) * (1.0 / _SCALE)
    dh = jnp.sum(m.reshape(_BN, _K, _H) * w[:, :, None], axis=1)
    x = _ln(hV_ref[...] + dh, g1_ref[...], be1_ref[...])
    f = _gelu(_dot(x, Win_ref[...]) + bin_ref[...])
    f = _dot(f, Wout_ref[...]) + bout_ref[...]
    x = _ln(x + f, g2_ref[...], be2_ref[...]) * mv_ref[...]
    hV_out_ref[...] = x
    A2_ref[...] = _dot(x, W11a_ref[...]) + b11_ref[...]
    P2_ref[...] = _dot(x, W11c_ref[...])


def _pass1(hV, hE, G1, A1, ma, cw, mv, weights):
    node = pl.BlockSpec((_BN, _H), lambda i: (i, 0))
    full = lambda s: pl.BlockSpec(s, lambda i: tuple(0 for _ in s))
    return pl.pallas_call(
        _pass1_body,
        grid=(_NBLK,),
        in_specs=[
            node,
            pl.BlockSpec((_BN, _K, _H), lambda i: (i, 0, 0)),
            pl.BlockSpec((_BN * _K, _H), lambda i: (i, 0)),
            node,
            pl.BlockSpec((_BN, _K), lambda i: (i, 0)),
            pl.BlockSpec((_BN, _K), lambda i: (i, 0)),
            pl.BlockSpec((_BN, 1), lambda i: (i, 0)),
        ] + [full(w.shape) for w in weights],
        out_specs=[node, node, node],
        out_shape=[jax.ShapeDtypeStruct((_N, _H), jnp.float32)] * 3,
        compiler_params=pltpu.CompilerParams(
            dimension_semantics=("arbitrary",)),
    )(hV, hE, G1, A1, ma, cw, mv, *weights)


def _pass2_body(hE_ref, G_ref, A2_ref,
                W11b_ref, W12_ref, b12_ref, W13_ref, b13_ref, g3_ref, be3_ref,
                out_ref):
    hE = hE_ref[...].reshape(_BN * _K, _H)
    t = _dot(hE, W11b_ref[...]) + G_ref[...]
    t = t.reshape(_BN, _K, _H) + A2_ref[...][:, None, :]
    m = _gelu(t).reshape(_BN * _K, _H)
    m = _gelu(_dot(m, W12_ref[...]) + b12_ref[...])
    m = _dot(m, W13_ref[...]) + b13_ref[...]
    out = _ln(hE + m, g3_ref[...], be3_ref[...])
    out_ref[...] = out.reshape(_BN, _K, _H)


def _pass2(hE, G2, A2, weights):
    full = lambda s: pl.BlockSpec(s, lambda i: tuple(0 for _ in s))
    return pl.pallas_call(
        _pass2_body,
        grid=(_NBLK,),
        in_specs=[
            pl.BlockSpec((_BN, _K, _H), lambda i: (i, 0, 0)),
            pl.BlockSpec((_BN * _K, _H), lambda i: (i, 0)),
            pl.BlockSpec((_BN, _H), lambda i: (i, 0)),
        ] + [full(w.shape) for w in weights],
        out_specs=pl.BlockSpec((_BN, _K, _H), lambda i: (i, 0, 0)),
        out_shape=jax.ShapeDtypeStruct((_N, _K, _H), jnp.float32),
        compiler_params=pltpu.CompilerParams(
            dimension_semantics=("arbitrary",)),
    )(hE, G2, A2, *weights)


def kernel(h_V, h_E, E_idx, mask_V, mask_attend, conf_weights, params):
    p = params
    hV = h_V.reshape(_N, _H)
    hE = h_E.reshape(_N, _K, _H)
    ma = mask_attend.reshape(_N, _K)
    cw = conf_weights.reshape(_N, _K)
    mv = mask_V.reshape(_N, 1)
    idx = E_idx.reshape(_N * _K).astype(jnp.int32)
    idx2d = jnp.pad(idx, (0, _NKP - _N * _K)).reshape(_NCHUNK, _CH)

    def rows(w):
        return w[:_H], w[_H:2 * _H], w[2 * _H:]

    W1a, W1b, W1c = rows(p['W1'][0])
    W11a, W11b, W11c = rows(p['W11'][0])
    b1 = p['W1'][1].reshape(1, _H)
    b11 = p['W11'][1].reshape(1, _H)
    W2, b2 = p['W2'][0], p['W2'][1].reshape(1, _H)
    W3, b3 = p['W3'][0], p['W3'][1].reshape(1, _H)
    W12, b12 = p['W12'][0], p['W12'][1].reshape(1, _H)
    W13, b13 = p['W13'][0], p['W13'][1].reshape(1, _H)
    Win, bin_ = p['Win'][0], p['Win'][1].reshape(1, 4 * _H)
    Wout, bout = p['Wout'][0], p['Wout'][1].reshape(1, _H)
    g1, be1 = (v.reshape(1, _H) for v in p['ln1'])
    g2, be2 = (v.reshape(1, _H) for v in p['ln2'])
    g3, be3 = (v.reshape(1, _H) for v in p['ln3'])

    A1, P1 = _prep(hV, W1a, b1, W1c)
    G1 = _sc_gather(P1, idx2d)
    hV2, A2, P2 = _pass1(
        hV, hE, G1, A1, ma, cw, mv,
        [W1b, W2, b2, W3, b3, g1, be1, Win, bin_, Wout, bout, g2, be2,
         W11a, b11, W11c])
    G2 = _sc_gather(P2, idx2d)
    hE2 = _pass2(hE, G2, A2, [W11b, W12, b12, W13, b13, g3, be3])
    return (hV2.reshape(1, _N, _H), hE2.reshape(1, _N, _K, _H))


# speedup vs baseline: 2.6091x; 1.0772x over previous
import functools
import math

import jax
import jax.numpy as jnp
from jax import lax
from jax.experimental import pallas as pl
from jax.experimental.pallas import tpu as pltpu
from jax.experimental.pallas import tpu_sc as plsc

_N, _K, _H = 10000, 32, 128
_SCALE = 30.0

_BN = 400
_NBLK = _N // _BN

_CH = 128
_NW = 32
_NCH_W = (-(-(_N * _K) // (_CH * _NW)) + 7) // 8 * 8
_NCHUNK = _NCH_W * _NW
_NKP = _NCHUNK * _CH


def _gelu(x):
    return 0.5 * x * (1.0 + lax.erf(x * (1.0 / math.sqrt(2.0))))


def _ln(x, g, b):
    mu = jnp.mean(x, axis=-1, keepdims=True)
    xc = x - mu
    var = jnp.mean(xc * xc, axis=-1, keepdims=True)
    return xc * lax.rsqrt(var + 1e-5) * g + b


def _dot(a, b):
    return jnp.dot(a.astype(jnp.bfloat16), b.astype(jnp.bfloat16),
                   preferred_element_type=jnp.float32)


def _prep_body(hV_ref, W1a_ref, b1_ref, W1c_ref, A1_ref, P1_ref):
    hV = hV_ref[...]
    A1_ref[...] = _dot(hV, W1a_ref[...]) + b1_ref[...]
    P1_ref[...] = _dot(hV, W1c_ref[...])


def _prep(hV, W1a, b1, W1c):
    return pl.pallas_call(
        _prep_body,
        out_shape=[jax.ShapeDtypeStruct((_N, _H), jnp.float32),
                   jax.ShapeDtypeStruct((_N, _H), jnp.float32)],
    )(hV, W1a, b1, W1c)


def _sc_gather(tbl, idx2d):
    mesh = plsc.VectorSubcoreMesh(core_axis_name="c", subcore_axis_name="s")

    @functools.partial(
        pl.kernel, mesh=mesh,
        out_type=jax.ShapeDtypeStruct((_NKP, _H), jnp.float32),
        scratch_types=[
            pltpu.VMEM((_NCH_W, _CH), jnp.int32),
            pltpu.VMEM((_CH, _H), jnp.float32),
            pltpu.VMEM((_CH, _H), jnp.float32),
            pltpu.VMEM_SHARED((_N, _H), jnp.float32),
            pltpu.SemaphoreType.DMA,
            pltpu.SemaphoreType.DMA,
        ],
    )
    def gath(tbl_hbm, idx_hbm, out_hbm, idxs, buf0, buf1,
             tblS, sem0, sem1):
        s = lax.axis_index("s")
        wid = s * 2 + lax.axis_index("c")
        c0 = wid * _NCH_W
        rowb = (buf0, buf1)
        semb = (sem0, sem1)

        @pl.when(s < 10)
        def _():
            r0 = pl.multiple_of(s * (_N // 10), 8)
            pltpu.sync_copy(tbl_hbm.at[pl.ds(r0, _N // 10)],
                            tblS.at[pl.ds(r0, _N // 10)])
        plsc.subcore_barrier()

        pltpu.sync_copy(idx_hbm.at[pl.ds(c0, _NCH_W)], idxs)

        def fire(j, b):
            pltpu.async_copy(tblS.at[idxs.at[j]], rowb[b], semb[b])

        def store(j, b):
            pltpu.make_async_copy(tblS.at[idxs.at[j]], rowb[b],
                                  semb[b]).wait()
            off = pl.multiple_of((c0 + j) * _CH, _CH)
            pltpu.sync_copy(rowb[b], out_hbm.at[pl.ds(off, _CH)])

        for b in range(2):
            fire(b, b)

        def body(q, carry):
            base = 2 * q
            for b in range(2):
                store(base + b, b)
                fire(base + b + 2, b)
            return carry

        lax.fori_loop(0, (_NCH_W - 2) // 2, body, 0)
        for b in range(2):
            store(_NCH_W - 2 + b, b)

    return gath(tbl, idx2d)


def _pass1_body(hV_ref, hE_ref, G_ref, A1_ref, ma_ref, cw_ref, mv_ref,
                W1b_ref, W2_ref, b2_ref, W3_ref, b3_ref, g1_ref, be1_ref,
                Win_ref, bin_ref, Wout_ref, bout_ref, g2_ref, be2_ref,
                W11a_ref, b11_ref, W11c_ref,
                hV_out_ref, A2_ref, P2_ref):
    hE = hE_ref[...].reshape(_BN * _K, _H)
    t = _dot(hE, W1b_ref[...]) + G_ref[...]
    t = t.reshape(_BN, _K, _H) + A1_ref[...][:, None, :]
    m = _gelu(t).reshape(_BN * _K, _H)
    m = _gelu(_dot(m, W2_ref[...]) + b2_ref[...])
    m = _dot(m, W3_ref[...]) + b3_ref[...]
    w = (ma_ref[...] * cw_ref[...]) * (1.0 / _SCALE)
    dh = jnp.sum(m.reshape(_BN, _K, _H) * w[:, :, None], axis=1)
    x = _ln(hV_ref[...] + dh, g1_ref[...], be1_ref[...])
    f = _gelu(_dot(x, Win_ref[...]) + bin_ref[...])
    f = _dot(f, Wout_ref[...]) + bout_ref[...]
    x = _ln(x + f, g2_ref[...], be2_ref[...]) * mv_ref[...]
    hV_out_ref[...] = x
    A2_ref[...] = _dot(x, W11a_ref[...]) + b11_ref[...]
    P2_ref[...] = _dot(x, W11c_ref[...])


def _pass1(hV, hE, G1, A1, ma, cw, mv, weights):
    node = pl.BlockSpec((_BN, _H), lambda i: (i, 0))
    full = lambda s: pl.BlockSpec(s, lambda i: tuple(0 for _ in s))
    return pl.pallas_call(
        _pass1_body,
        grid=(_NBLK,),
        in_specs=[
            node,
            pl.BlockSpec((_BN, _K, _H), lambda i: (i, 0, 0)),
            pl.BlockSpec((_BN * _K, _H), lambda i: (i, 0)),
            node,
            pl.BlockSpec((_BN, _K), lambda i: (i, 0)),
            pl.BlockSpec((_BN, _K), lambda i: (i, 0)),
            pl.BlockSpec((_BN, 1), lambda i: (i, 0)),
        ] + [full(w.shape) for w in weights],
        out_specs=[node, node, node],
        out_shape=[jax.ShapeDtypeStruct((_N, _H), jnp.float32)] * 3,
        compiler_params=pltpu.CompilerParams(
            dimension_semantics=("arbitrary",)),
    )(hV, hE, G1, A1, ma, cw, mv, *weights)


def _pass2_body(hE_ref, G_ref, A2_ref,
                W11b_ref, W12_ref, b12_ref, W13_ref, b13_ref, g3_ref, be3_ref,
                out_ref):
    hE = hE_ref[...].reshape(_BN * _K, _H)
    t = _dot(hE, W11b_ref[...]) + G_ref[...]
    t = t.reshape(_BN, _K, _H) + A2_ref[...][:, None, :]
    m = _gelu(t).reshape(_BN * _K, _H)
    m = _gelu(_dot(m, W12_ref[...]) + b12_ref[...])
    m = _dot(m, W13_ref[...]) + b13_ref[...]
    out = _ln(hE + m, g3_ref[...], be3_ref[...])
    out_ref[...] = out.reshape(_BN, _K, _H)


def _pass2(hE, G2, A2, weights):
    full = lambda s: pl.BlockSpec(s, lambda i: tuple(0 for _ in s))
    return pl.pallas_call(
        _pass2_body,
        grid=(_NBLK,),
        in_specs=[
            pl.BlockSpec((_BN, _K, _H), lambda i: (i, 0, 0)),
            pl.BlockSpec((_BN * _K, _H), lambda i: (i, 0)),
            pl.BlockSpec((_BN, _H), lambda i: (i, 0)),
        ] + [full(w.shape) for w in weights],
        out_specs=pl.BlockSpec((_BN, _K, _H), lambda i: (i, 0, 0)),
        out_shape=jax.ShapeDtypeStruct((_N, _K, _H), jnp.float32),
        compiler_params=pltpu.CompilerParams(
            dimension_semantics=("arbitrary",)),
    )(hE, G2, A2, *weights)


def kernel(h_V, h_E, E_idx, mask_V, mask_attend, conf_weights, params):
    p = params
    hV = h_V.reshape(_N, _H)
    hE = h_E.reshape(_N, _K, _H)
    ma = mask_attend.reshape(_N, _K)
    cw = conf_weights.reshape(_N, _K)
    mv = mask_V.reshape(_N, 1)
    idx = E_idx.reshape(_N * _K).astype(jnp.int32)
    idx2d = jnp.pad(idx, (0, _NKP - _N * _K)).reshape(_NCHUNK, _CH)

    def rows(w):
        return w[:_H], w[_H:2 * _H], w[2 * _H:]

    W1a, W1b, W1c = rows(p['W1'][0])
    W11a, W11b, W11c = rows(p['W11'][0])
    b1 = p['W1'][1].reshape(1, _H)
    b11 = p['W11'][1].reshape(1, _H)
    W2, b2 = p['W2'][0], p['W2'][1].reshape(1, _H)
    W3, b3 = p['W3'][0], p['W3'][1].reshape(1, _H)
    W12, b12 = p['W12'][0], p['W12'][1].reshape(1, _H)
    W13, b13 = p['W13'][0], p['W13'][1].reshape(1, _H)
    Win, bin_ = p['Win'][0], p['Win'][1].reshape(1, 4 * _H)
    Wout, bout = p['Wout'][0], p['Wout'][1].reshape(1, _H)
    g1, be1 = (v.reshape(1, _H) for v in p['ln1'])
    g2, be2 = (v.reshape(1, _H) for v in p['ln2'])
    g3, be3 = (v.reshape(1, _H) for v in p['ln3'])

    A1, P1 = _prep(hV, W1a, b1, W1c)
    G1 = _sc_gather(P1, idx2d)
    hV2, A2, P2 = _pass1(
        hV, hE, G1, A1, ma, cw, mv,
        [W1b, W2, b2, W3, b3, g1, be1, Win, bin_, Wout, bout, g2, be2,
         W11a, b11, W11c])
    G2 = _sc_gather(P2, idx2d)
    hE2 = _pass2(hE, G2, A2, [W11b, W12, b12, W13, b13, g3, be3])
    return (hV2.reshape(1, _N, _H), hE2.reshape(1, _N, _K, _H))
